# Initial kernel scaffold; baseline (speedup 1.0000x reference)
import jax
import jax.numpy as jnp
from jax.experimental import pallas as pl


def kernel(x_indices, x_values, W_freq, b_freq, W_rare1, b_rare1, W_rare2, W_mid, b_mid, W_last, b_last):
    raise NotImplementedError("write your pallas kernel here")


# speedup vs baseline: 3.8312x; 3.8312x over previous
import functools

import jax
import jax.numpy as jnp
from jax import lax
from jax.experimental import pallas as pl
from jax.experimental.pallas import tpu as pltpu
from jax.experimental.pallas import tpu_sc as plsc

BATCH = 1024
IN_SIZE = 100000
FREQ = 90000
HID0 = 256
TAIL = 64
TAILP = 128
HID1 = 128
OUT = 1000
NNZ = 131072

NC = 2
NS = 16
NW = NC * NS
NQ = 4
NR = NW // NQ
QROWS = BATCH // NQ
PER_R = NNZ // NR
STAGE = 2048
GB = 64
QCAP = 160
PAD_IDX_MASK = 8191


def _queue_append(qcol, qrow, qval, off, c, rl, v, m):
    sl = pl.ds(off, 16)
    plsc.store_compressed(qcol.at[sl], c, mask=m)
    plsc.store_compressed(qrow.at[sl], rl, mask=m)
    plsc.store_compressed(qval.at[sl], v, mask=m)
    return off + plsc.all_reduce_population_count(m)[0]


def _queue_shift(qcol, qrow, qval):
    for k in range(6):
        dst = pl.ds(16 * k, 16)
        src = pl.ds(GB + 16 * k, 16)
        qcol[dst] = qcol[src]
        qrow[dst] = qrow[src]
        qval[dst] = qval[src]


def _queue_pad(qcol, qrow, qval, off):
    lanes = lax.iota(jnp.int32, 16)
    for k in range(QCAP // 16):
        sl = pl.ds(16 * k, 16)
        lane_ids = lanes + (16 * k)
        m = lane_ids >= off
        pad_c = lane_ids & PAD_IDX_MASK
        qcol[sl] = jnp.where(m, pad_c, qcol[sl])
        qrow[sl] = jnp.where(m, 0, qrow[sl])
        qval[sl] = jnp.where(m, 0.0, qval[sl])


def _fire(table_hbm, qcol, qrow, qval, buf, acc, sem, n_vregs):
    cp = pltpu.async_copy(table_hbm.at[qcol.at[pl.ds(0, GB)]], buf, sem)
    cp.wait()

    def g_body(g2, _):
        b = g2 * 16
        v16 = qval[pl.ds(b, 16)]
        r16 = qrow[pl.ds(b, 16)]
        for t in range(16):
            j = b + t
            vv = jnp.full((16,), v16[t], jnp.float32)
            rl = r16[t]
            for k in range(n_vregs):
                s = pl.ds(16 * k, 16)
                acc[rl, s] = acc[rl, s] + buf[j, s] * vv
        return _

    lax.fori_loop(0, GB // 16, g_body, None)


def _sc_embed_body(cols_hbm, rows_hbm, vals_hbm, wf_hbm, wr_hbm,
                   hf_out, hr_out,
                   colb, rowb, valb,
                   fqc, fqr, fqv, rqc, rqr, rqv,
                   fbuf, rbuf, acc_f, acc_r, semi, semf, semr):
    cid = lax.axis_index("c")
    sid = lax.axis_index("s")
    wid = cid * NS + sid
    rb = wid // NQ
    q = wid % NQ
    qlo = q * QROWS

    zf = jnp.zeros((16,), jnp.float32)

    def _zero_body(j, _):
        for k in range(HID0 // 16):
            acc_f[j, pl.ds(16 * k, 16)] = zf
        for k in range(TAIL // 16):
            acc_r[j, pl.ds(16 * k, 16)] = zf
        return _
    lax.fori_loop(0, QROWS, _zero_body, None)

    def _stage_body(si, offs):
        base = rb * PER_R + si * STAGE
        pltpu.sync_copy(cols_hbm.at[pl.ds(base, STAGE)], colb)
        pltpu.sync_copy(rows_hbm.at[pl.ds(base, STAGE)], rowb)
        pltpu.sync_copy(vals_hbm.at[pl.ds(base, STAGE)], valb)

        def _scan_body(gi, offs):
            off_f, off_r = offs
            sl = pl.ds(gi * 16, 16)
            c = colb[sl]
            r = rowb[sl]
            v = valb[sl]
            rl = r - qlo
            inq = (r >= qlo) & (r < qlo + QROWS)
            isf = c < FREQ
            mf = inq & isf
            mr = inq & (~isf)
            off_f = _queue_append(fqc, fqr, fqv, off_f, c, rl, v, mf)
            off_r = _queue_append(rqc, rqr, rqv, off_r, c - FREQ, rl, v, mr)

            @pl.when(off_f >= GB)
            def _():
                _fire(wf_hbm, fqc, fqr, fqv, fbuf, acc_f, semf, HID0 // 16)
                _queue_shift(fqc, fqr, fqv)

            off_f = jnp.where(off_f >= GB, off_f - GB, off_f)

            @pl.when(off_r >= GB)
            def _():
                _fire(wr_hbm, rqc, rqr, rqv, rbuf, acc_r, semr, TAIL // 16)
                _queue_shift(rqc, rqr, rqv)

            off_r = jnp.where(off_r >= GB, off_r - GB, off_r)
            return (off_f, off_r)

        return lax.fori_loop(0, STAGE // 16, _scan_body, offs)

    off_f, off_r = lax.fori_loop(0, PER_R // STAGE, _stage_body,
                                 (jnp.int32(0), jnp.int32(0)))

    _queue_pad(fqc, fqr, fqv, off_f)
    _queue_pad(rqc, rqr, rqv, off_r)

    def _drain_body(i, _):
        _fire(wf_hbm, fqc, fqr, fqv, fbuf, acc_f, semf, HID0 // 16)
        _queue_shift(fqc, fqr, fqv)
        _fire(wr_hbm, rqc, rqr, rqv, rbuf, acc_r, semr, TAIL // 16)
        _queue_shift(rqc, rqr, rqv)
        return _
    lax.fori_loop(0, 2, _drain_body, None)

    pos = (q * NR + rb) * QROWS
    pltpu.sync_copy(acc_f, hf_out.at[pl.ds(pos, QROWS)])
    pltpu.sync_copy(acc_r, hr_out.at[pl.ds(pos, QROWS)])


_sc_embed = functools.partial(
    pl.kernel,
    out_type=(
        jax.ShapeDtypeStruct((NW * QROWS, HID0), jnp.float32),
        jax.ShapeDtypeStruct((NW * QROWS, TAIL), jnp.float32),
    ),
    mesh=plsc.VectorSubcoreMesh(core_axis_name="c", subcore_axis_name="s"),
    compiler_params=pltpu.CompilerParams(needs_layout_passes=False),
    scratch_types=(
        pltpu.VMEM((STAGE,), jnp.int32),
        pltpu.VMEM((STAGE,), jnp.int32),
        pltpu.VMEM((STAGE,), jnp.float32),
        pltpu.VMEM((QCAP,), jnp.int32),
        pltpu.VMEM((QCAP,), jnp.int32),
        pltpu.VMEM((QCAP,), jnp.float32),
        pltpu.VMEM((QCAP,), jnp.int32),
        pltpu.VMEM((QCAP,), jnp.int32),
        pltpu.VMEM((QCAP,), jnp.float32),
        pltpu.VMEM((GB, HID0), jnp.float32),
        pltpu.VMEM((GB, TAILP), jnp.float32),
        pltpu.VMEM((QROWS, HID0), jnp.float32),
        pltpu.VMEM((QROWS, TAIL), jnp.float32),
        pltpu.SemaphoreType.DMA,
        pltpu.SemaphoreType.DMA,
        pltpu.SemaphoreType.DMA,
    ),
)(_sc_embed_body)


def _dense_body(hfp, hrp, bf, br1, wr2, wm, bm, wl, bl, out_ref):
    hf_qs = []
    hr_qs = []
    for q in range(NQ):
        hf_q = hfp[pl.ds(q * NR * QROWS, QROWS), :]
        hr_q = hrp[pl.ds(q * NR * QROWS, QROWS), :]
        for r in range(1, NR):
            hf_q = hf_q + hfp[pl.ds((q * NR + r) * QROWS, QROWS), :]
            hr_q = hr_q + hrp[pl.ds((q * NR + r) * QROWS, QROWS), :]
        hf_qs.append(hf_q)
        hr_qs.append(hr_q)
    hf = jnp.concatenate(hf_qs, axis=0)
    hr = jnp.concatenate(hr_qs, axis=0) + br1[0:1, :]
    h0 = hf + bf[0:1, :] + lax.dot(
        hr, wr2[...], precision=lax.Precision.HIGHEST,
        preferred_element_type=jnp.float32)
    h1 = lax.dot(jnp.maximum(h0, 0.0), wm[...],
                 precision=lax.Precision.HIGHEST,
                 preferred_element_type=jnp.float32) + bm[0:1, :]
    out_ref[...] = lax.dot(jnp.maximum(h1, 0.0), wl[...],
                           precision=lax.Precision.HIGHEST,
                           preferred_element_type=jnp.float32) + bl[0:1, :]


_dense_call = pl.pallas_call(
    _dense_body,
    out_shape=jax.ShapeDtypeStruct((BATCH, OUT), jnp.float32),
)


def kernel(x_indices, x_values, W_freq, b_freq, W_rare1, b_rare1, W_rare2,
           W_mid, b_mid, W_last, b_last):
    rows = x_indices[0]
    cols = x_indices[1]
    W_rare1p = jnp.pad(W_rare1, ((0, 0), (0, TAILP - TAIL)))
    hf2, hr2 = _sc_embed(cols, rows, x_values, W_freq, W_rare1p)
    return _dense_call(
        hf2, hr2,
        b_freq.reshape(1, HID0), b_rare1.reshape(1, TAIL), W_rare2,
        W_mid, b_mid.reshape(1, HID1), W_last, b_last.reshape(1, OUT))


# speedup vs baseline: 4.2742x; 1.1156x over previous
import functools

import jax
import jax.numpy as jnp
from jax import lax
from jax.experimental import pallas as pl
from jax.experimental.pallas import tpu as pltpu
from jax.experimental.pallas import tpu_sc as plsc

BATCH = 1024
IN_SIZE = 100000
FREQ = 90000
HID0 = 256
TAIL = 64
TAILP = 128
HID1 = 128
OUT = 1000
NNZ = 131072

NC = 2
NS = 16
NW = NC * NS
NQ = 4
NR = NW // NQ
QROWS = BATCH // NQ
PER_R = NNZ // NR
STAGE = 512
GB = 32
QCAP = 112
PAD_IDX_MASK = 8191


def _i16(v):
    return jnp.full((16,), v, jnp.int32)


def _queue_append(qcol, qrow, qval, off, c, rl, v, m):
    sl = pl.ds(off, 16)
    plsc.store_compressed(qcol.at[sl], c, mask=m)
    plsc.store_compressed(qrow.at[sl], rl, mask=m)
    plsc.store_compressed(qval.at[sl], v, mask=m)
    return off + plsc.all_reduce_population_count(m)[0]


def _queue_shift(qcol, qrow, qval):
    for k in range((QCAP - GB) // 16):
        dst = pl.ds(16 * k, 16)
        src = pl.ds(GB + 16 * k, 16)
        qcol[dst] = qcol[src]
        qrow[dst] = qrow[src]
        qval[dst] = qval[src]


def _queue_pad(qcol, qrow, qval, off):
    lanes = lax.iota(jnp.int32, 16)
    for k in range(QCAP // 16):
        sl = pl.ds(16 * k, 16)
        lane_ids = lanes + (16 * k)
        m = lane_ids >= off
        pad_c = lane_ids & PAD_IDX_MASK
        qcol[sl] = jnp.where(m, pad_c, qcol[sl])
        qrow[sl] = jnp.where(m, 0, qrow[sl])
        qval[sl] = jnp.where(m, 0.0, qval[sl])


def _sc_embed_body(cols_hbm, rows_hbm, vals_hbm, wf_hbm, wr_hbm,
                   hf_out, hr_out,
                   colb, rowb, valb,
                   fqc, fqr, fqv, rqc, rqr, rqv,
                   gcol, grow, gval,
                   fbuf, rbuf, acc_f, acc_r, semf, semr):
    cid = lax.axis_index("c")
    sid = lax.axis_index("s")
    wid = cid * NS + sid
    rb = wid // NQ
    q = wid % NQ
    qlo = q * QROWS
    lanes = lax.iota(jnp.int32, 16)

    zf = jnp.zeros((16,), jnp.float32)

    def _zero_body(j, _):
        for k in range(HID0 // 16):
            acc_f[j, pl.ds(16 * k, 16)] = zf
        for k in range(TAIL // 16):
            acc_r[j, pl.ds(16 * k, 16)] = zf
        return _
    lax.fori_loop(0, QROWS, _zero_body, None)

    def _issue_fire(p):
        for k in range(GB // 16):
            sl = pl.ds(16 * k, 16)
            gcol[p, sl] = fqc[sl]
            grow[p, sl] = fqr[sl]
            gval[p, sl] = fqv[sl]
        pltpu.async_copy(wf_hbm.at[gcol.at[p]], fbuf.at[p], semf.at[p])
        _queue_shift(fqc, fqr, fqv)

    def _acc_fire(p):
        pltpu.make_async_copy(wf_hbm.at[gcol.at[p]], fbuf.at[p],
                              semf.at[p]).wait()

        def body(j, _):
            vv = plsc.load_gather(gval, [_i16(p), _i16(j)])
            rl = plsc.load_gather(grow, [_i16(p), _i16(j)])
            for k in range(HID0 // 16):
                x = fbuf[p, j, pl.ds(16 * k, 16)] * vv
                plsc.addupdate_scatter(acc_f, [rl, lanes + 16 * k], x)
            return _
        lax.fori_loop(0, GB, body, None)

    def _rare_fire():
        pltpu.async_copy(wr_hbm.at[rqc.at[pl.ds(0, GB)]], rbuf, semr).wait()

        def body(j, _):
            vv = plsc.load_gather(rqv, [_i16(j)])
            rl = plsc.load_gather(rqr, [_i16(j)])
            for k in range(TAIL // 16):
                x = rbuf[j, pl.ds(16 * k, 16)] * vv
                plsc.addupdate_scatter(acc_r, [rl, lanes + 16 * k], x)
            return _
        lax.fori_loop(0, GB, body, None)
        _queue_shift(rqc, rqr, rqv)

    def _stage_body(si, carry):
        base = rb * PER_R + si * STAGE
        pltpu.sync_copy(cols_hbm.at[pl.ds(base, STAGE)], colb)
        pltpu.sync_copy(rows_hbm.at[pl.ds(base, STAGE)], rowb)
        pltpu.sync_copy(vals_hbm.at[pl.ds(base, STAGE)], valb)

        def _scan_body(gi, carry):
            off_f, off_r, p, pend = carry
            sl = pl.ds(gi * 16, 16)
            c = colb[sl]
            r = rowb[sl]
            v = valb[sl]
            rl = r - qlo
            inq = (r >= qlo) & (r < qlo + QROWS)
            isf = c < FREQ
            mf = inq & isf
            mr = inq & (~isf)
            off_f = _queue_append(fqc, fqr, fqv, off_f, c, rl, v, mf)
            off_r = _queue_append(rqc, rqr, rqv, off_r, c - FREQ, rl, v, mr)

            fire = off_f >= GB

            @pl.when(fire)
            def _():
                _issue_fire(p)

                @pl.when(pend == 1)
                def _():
                    _acc_fire(1 - p)

            off_f = jnp.where(fire, off_f - GB, off_f)
            p = jnp.where(fire, 1 - p, p)
            pend = jnp.where(fire, 1, pend)

            fire_r = off_r >= GB

            @pl.when(fire_r)
            def _():
                _rare_fire()

            off_r = jnp.where(fire_r, off_r - GB, off_r)
            return (off_f, off_r, p, pend)

        return lax.fori_loop(0, STAGE // 16, _scan_body, carry)

    off_f, off_r, p, pend = lax.fori_loop(
        0, PER_R // STAGE, _stage_body,
        (jnp.int32(0), jnp.int32(0), jnp.int32(0), jnp.int32(0)))

    @pl.when(pend == 1)
    def _():
        _acc_fire(1 - p)

    _queue_pad(fqc, fqr, fqv, off_f)
    _queue_pad(rqc, rqr, rqv, off_r)

    def _drain_body(i, pp):
        _issue_fire(pp)
        _acc_fire(pp)
        _rare_fire()
        return 1 - pp
    lax.fori_loop(0, 2, _drain_body, p)

    pos = (q * NR + rb) * QROWS
    pltpu.sync_copy(acc_f, hf_out.at[pl.ds(pos, QROWS)])
    pltpu.sync_copy(acc_r, hr_out.at[pl.ds(pos, QROWS)])


_sc_embed = functools.partial(
    pl.kernel,
    out_type=(
        jax.ShapeDtypeStruct((NW * QROWS, HID0), jnp.float32),
        jax.ShapeDtypeStruct((NW * QROWS, TAIL), jnp.float32),
    ),
    mesh=plsc.VectorSubcoreMesh(core_axis_name="c", subcore_axis_name="s"),
    compiler_params=pltpu.CompilerParams(needs_layout_passes=False),
    scratch_types=(
        pltpu.VMEM((STAGE,), jnp.int32),
        pltpu.VMEM((STAGE,), jnp.int32),
        pltpu.VMEM((STAGE,), jnp.float32),
        pltpu.VMEM((QCAP,), jnp.int32),
        pltpu.VMEM((QCAP,), jnp.int32),
        pltpu.VMEM((QCAP,), jnp.float32),
        pltpu.VMEM((QCAP,), jnp.int32),
        pltpu.VMEM((QCAP,), jnp.int32),
        pltpu.VMEM((QCAP,), jnp.float32),
        pltpu.VMEM((2, GB), jnp.int32),
        pltpu.VMEM((2, GB), jnp.int32),
        pltpu.VMEM((2, GB), jnp.float32),
        pltpu.VMEM((2, GB, HID0), jnp.float32),
        pltpu.VMEM((GB, TAILP), jnp.float32),
        pltpu.VMEM((QROWS, HID0), jnp.float32),
        pltpu.VMEM((QROWS, TAIL), jnp.float32),
        pltpu.SemaphoreType.DMA((2,)),
        pltpu.SemaphoreType.DMA,
    ),
)(_sc_embed_body)


def _dense_body(hfp, hrp, bf, br1, wr2, wm, bm, wl, bl, out_ref):
    hf_qs = []
    hr_qs = []
    for q in range(NQ):
        hf_q = hfp[pl.ds(q * NR * QROWS, QROWS), :]
        hr_q = hrp[pl.ds(q * NR * QROWS, QROWS), :]
        for r in range(1, NR):
            hf_q = hf_q + hfp[pl.ds((q * NR + r) * QROWS, QROWS), :]
            hr_q = hr_q + hrp[pl.ds((q * NR + r) * QROWS, QROWS), :]
        hf_qs.append(hf_q)
        hr_qs.append(hr_q)
    hf = jnp.concatenate(hf_qs, axis=0)
    hr = jnp.concatenate(hr_qs, axis=0) + br1[0:1, :]
    h0 = hf + bf[0:1, :] + lax.dot(
        hr, wr2[...], precision=lax.Precision.HIGHEST,
        preferred_element_type=jnp.float32)
    h1 = lax.dot(jnp.maximum(h0, 0.0), wm[...],
                 precision=lax.Precision.HIGHEST,
                 preferred_element_type=jnp.float32) + bm[0:1, :]
    out_ref[...] = lax.dot(jnp.maximum(h1, 0.0), wl[...],
                           precision=lax.Precision.HIGHEST,
                           preferred_element_type=jnp.float32) + bl[0:1, :]


_dense_call = pl.pallas_call(
    _dense_body,
    out_shape=jax.ShapeDtypeStruct((BATCH, OUT), jnp.float32),
)


def kernel(x_indices, x_values, W_freq, b_freq, W_rare1, b_rare1, W_rare2,
           W_mid, b_mid, W_last, b_last):
    rows = x_indices[0]
    cols = x_indices[1]
    W_rare1p = jnp.pad(W_rare1, ((0, 0), (0, TAILP - TAIL)))
    hf2, hr2 = _sc_embed(cols, rows, x_values, W_freq, W_rare1p)
    return _dense_call(
        hf2, hr2,
        b_freq.reshape(1, HID0), b_rare1.reshape(1, TAIL), W_rare2,
        W_mid, b_mid.reshape(1, HID1), W_last, b_last.reshape(1, OUT))


# speedup vs baseline: 7.7767x; 1.8195x over previous
import functools

import jax
import jax.numpy as jnp
from jax import lax
from jax.experimental import pallas as pl
from jax.experimental.pallas import tpu as pltpu
from jax.experimental.pallas import tpu_sc as plsc

BATCH = 1024
IN_SIZE = 100000
FREQ = 90000
HID0 = 256
TAIL = 64
TAILP = 128
HID1 = 128
OUT = 1000
NNZ = 131072

NC = 2
NS = 16
NW = NC * NS
NQ = 4
NR = NW // NQ
QROWS = BATCH // NQ
PER_R = NNZ // NR
STAGE = 512
GB = 32
QCAP = 112
PAD_IDX_MASK = 8191


def _i16(v):
    return jnp.full((16,), v, jnp.int32)


def _queue_append(qcol, qrow, qval, off, c, rl, v, m):
    sl = pl.ds(off, 16)
    plsc.store_compressed(qcol.at[sl], c, mask=m)
    plsc.store_compressed(qrow.at[sl], rl, mask=m)
    plsc.store_compressed(qval.at[sl], v, mask=m)
    return off + plsc.all_reduce_population_count(m)[0]


def _queue_shift(qcol, qrow, qval):
    for k in range((QCAP - GB) // 16):
        dst = pl.ds(16 * k, 16)
        src = pl.ds(GB + 16 * k, 16)
        qcol[dst] = qcol[src]
        qrow[dst] = qrow[src]
        qval[dst] = qval[src]


def _queue_pad(qcol, qrow, qval, off):
    lanes = lax.iota(jnp.int32, 16)
    for k in range(QCAP // 16):
        sl = pl.ds(16 * k, 16)
        lane_ids = lanes + (16 * k)
        m = lane_ids >= off
        pad_c = lane_ids & PAD_IDX_MASK
        qcol[sl] = jnp.where(m, pad_c, qcol[sl])
        qrow[sl] = jnp.where(m, 0, qrow[sl])
        qval[sl] = jnp.where(m, 0.0, qval[sl])


def _sc_embed_body(cols_hbm, rows_hbm, vals_hbm, wf_hbm, wr_hbm,
                   hf_out, hr_out,
                   colb, rowb, valb,
                   fqc, fqr, fqv, rqc, rqr, rqv,
                   gcol, grow, gval,
                   fbuf, rbuf, acc_f, acc_r, semf, semr):
    cid = lax.axis_index("c")
    sid = lax.axis_index("s")
    wid = cid * NS + sid
    rb = wid // NQ
    q = wid % NQ
    qlo = q * QROWS
    lanes = lax.iota(jnp.int32, 16)

    zf = jnp.zeros((16,), jnp.float32)

    def _zero_body(j, _):
        for k in range(HID0 // 16):
            acc_f[j, pl.ds(16 * k, 16)] = zf
        for k in range(TAIL // 16):
            acc_r[j, pl.ds(16 * k, 16)] = zf
        return _
    lax.fori_loop(0, QROWS, _zero_body, None)

    def _issue_fire(p):
        for k in range(GB // 16):
            sl = pl.ds(16 * k, 16)
            gcol[p, sl] = fqc[sl]
            grow[p, sl] = fqr[sl]
            gval[p, sl] = fqv[sl]
        pltpu.async_copy(wf_hbm.at[gcol.at[p]], fbuf.at[p], semf.at[p])
        _queue_shift(fqc, fqr, fqv)

    def _acc_fire(p):
        pltpu.make_async_copy(wf_hbm.at[gcol.at[p]], fbuf.at[p],
                              semf.at[p]).wait()

        @plsc.parallel_loop(0, GB, unroll=4)
        def body(j):
            vv = plsc.load_gather(gval, [_i16(p), _i16(j)])
            rl = plsc.load_gather(grow, [_i16(p), _i16(j)])
            for k in range(HID0 // 16):
                x = fbuf[p, j, pl.ds(16 * k, 16)] * vv
                plsc.addupdate_scatter(acc_f, [rl, lanes + 16 * k], x)

    def _rare_fire():
        pltpu.async_copy(wr_hbm.at[rqc.at[pl.ds(0, GB)]], rbuf, semr).wait()

        @plsc.parallel_loop(0, GB, unroll=2)
        def body(j):
            vv = plsc.load_gather(rqv, [_i16(j)])
            rl = plsc.load_gather(rqr, [_i16(j)])
            for k in range(TAIL // 16):
                x = rbuf[j, pl.ds(16 * k, 16)] * vv
                plsc.addupdate_scatter(acc_r, [rl, lanes + 16 * k], x)
        _queue_shift(rqc, rqr, rqv)

    def _stage_body(si, carry):
        base = rb * PER_R + si * STAGE
        pltpu.sync_copy(cols_hbm.at[pl.ds(base, STAGE)], colb)
        pltpu.sync_copy(rows_hbm.at[pl.ds(base, STAGE)], rowb)
        pltpu.sync_copy(vals_hbm.at[pl.ds(base, STAGE)], valb)

        def _scan_body(gi, carry):
            off_f, off_r, p, pend = carry
            sl = pl.ds(gi * 16, 16)
            c = colb[sl]
            r = rowb[sl]
            v = valb[sl]
            rl = r - qlo
            inq = (r >= qlo) & (r < qlo + QROWS)
            isf = c < FREQ
            mf = inq & isf
            mr = inq & (~isf)
            off_f = _queue_append(fqc, fqr, fqv, off_f, c, rl, v, mf)
            off_r = _queue_append(rqc, rqr, rqv, off_r, c - FREQ, rl, v, mr)

            fire = off_f >= GB

            @pl.when(fire)
            def _():
                _issue_fire(p)

                @pl.when(pend == 1)
                def _():
                    _acc_fire(1 - p)

            off_f = jnp.where(fire, off_f - GB, off_f)
            p = jnp.where(fire, 1 - p, p)
            pend = jnp.where(fire, 1, pend)

            fire_r = off_r >= GB

            @pl.when(fire_r)
            def _():
                _rare_fire()

            off_r = jnp.where(fire_r, off_r - GB, off_r)
            return (off_f, off_r, p, pend)

        return lax.fori_loop(0, STAGE // 16, _scan_body, carry)

    off_f, off_r, p, pend = lax.fori_loop(
        0, PER_R // STAGE, _stage_body,
        (jnp.int32(0), jnp.int32(0), jnp.int32(0), jnp.int32(0)))

    @pl.when(pend == 1)
    def _():
        _acc_fire(1 - p)

    _queue_pad(fqc, fqr, fqv, off_f)
    _queue_pad(rqc, rqr, rqv, off_r)

    def _drain_body(i, pp):
        _issue_fire(pp)
        _acc_fire(pp)
        _rare_fire()
        return 1 - pp
    lax.fori_loop(0, 2, _drain_body, p)

    pos = (q * NR + rb) * QROWS
    pltpu.sync_copy(acc_f, hf_out.at[pl.ds(pos, QROWS)])
    pltpu.sync_copy(acc_r, hr_out.at[pl.ds(pos, QROWS)])


_sc_embed = functools.partial(
    pl.kernel,
    out_type=(
        jax.ShapeDtypeStruct((NW * QROWS, HID0), jnp.float32),
        jax.ShapeDtypeStruct((NW * QROWS, TAIL), jnp.float32),
    ),
    mesh=plsc.VectorSubcoreMesh(core_axis_name="c", subcore_axis_name="s"),
    compiler_params=pltpu.CompilerParams(needs_layout_passes=False),
    scratch_types=(
        pltpu.VMEM((STAGE,), jnp.int32),
        pltpu.VMEM((STAGE,), jnp.int32),
        pltpu.VMEM((STAGE,), jnp.float32),
        pltpu.VMEM((QCAP,), jnp.int32),
        pltpu.VMEM((QCAP,), jnp.int32),
        pltpu.VMEM((QCAP,), jnp.float32),
        pltpu.VMEM((QCAP,), jnp.int32),
        pltpu.VMEM((QCAP,), jnp.int32),
        pltpu.VMEM((QCAP,), jnp.float32),
        pltpu.VMEM((2, GB), jnp.int32),
        pltpu.VMEM((2, GB), jnp.int32),
        pltpu.VMEM((2, GB), jnp.float32),
        pltpu.VMEM((2, GB, HID0), jnp.float32),
        pltpu.VMEM((GB, TAILP), jnp.float32),
        pltpu.VMEM((QROWS, HID0), jnp.float32),
        pltpu.VMEM((QROWS, TAIL), jnp.float32),
        pltpu.SemaphoreType.DMA((2,)),
        pltpu.SemaphoreType.DMA,
    ),
)(_sc_embed_body)


def _dense_body(hfp, hrp, bf, br1, wr2, wm, bm, wl, bl, out_ref):
    hf_qs = []
    hr_qs = []
    for q in range(NQ):
        hf_q = hfp[pl.ds(q * NR * QROWS, QROWS), :]
        hr_q = hrp[pl.ds(q * NR * QROWS, QROWS), :]
        for r in range(1, NR):
            hf_q = hf_q + hfp[pl.ds((q * NR + r) * QROWS, QROWS), :]
            hr_q = hr_q + hrp[pl.ds((q * NR + r) * QROWS, QROWS), :]
        hf_qs.append(hf_q)
        hr_qs.append(hr_q)
    hf = jnp.concatenate(hf_qs, axis=0)
    hr = jnp.concatenate(hr_qs, axis=0) + br1[0:1, :]
    h0 = hf + bf[0:1, :] + lax.dot(
        hr, wr2[...], precision=lax.Precision.HIGHEST,
        preferred_element_type=jnp.float32)
    h1 = lax.dot(jnp.maximum(h0, 0.0), wm[...],
                 precision=lax.Precision.HIGHEST,
                 preferred_element_type=jnp.float32) + bm[0:1, :]
    out_ref[...] = lax.dot(jnp.maximum(h1, 0.0), wl[...],
                           precision=lax.Precision.HIGHEST,
                           preferred_element_type=jnp.float32) + bl[0:1, :]


_dense_call = pl.pallas_call(
    _dense_body,
    out_shape=jax.ShapeDtypeStruct((BATCH, OUT), jnp.float32),
)


def kernel(x_indices, x_values, W_freq, b_freq, W_rare1, b_rare1, W_rare2,
           W_mid, b_mid, W_last, b_last):
    rows = x_indices[0]
    cols = x_indices[1]
    W_rare1p = jnp.pad(W_rare1, ((0, 0), (0, TAILP - TAIL)))
    hf2, hr2 = _sc_embed(cols, rows, x_values, W_freq, W_rare1p)
    return _dense_call(
        hf2, hr2,
        b_freq.reshape(1, HID0), b_rare1.reshape(1, TAIL), W_rare2,
        W_mid, b_mid.reshape(1, HID1), W_last, b_last.reshape(1, OUT))
